# baseline (speedup 1.0000x reference)
import functools

import jax
import jax.numpy as jnp
from jax import lax
from jax.experimental import pallas as pl
from jax.experimental.pallas import tpu as pltpu
from jax.experimental.pallas import tpu_sc as plsc

NPAD = 10240
GSEG = 128
BLK = 512
WA = 64
WB = 128 - WA


def _sc_count(dstR, nc, ns, rpw):
    mesh = plsc.VectorSubcoreMesh(core_axis_name="c", subcore_axis_name="s")
    rows_per_tile = NPAD // ns

    @functools.partial(
        pl.kernel,
        out_type=jax.ShapeDtypeStruct((nc, NPAD), jnp.float32),
        mesh=mesh,
        scratch_types=[
            pltpu.VMEM((rpw, 128), jnp.int32),
            pltpu.VMEM((128,), jnp.float32),
            pltpu.VMEM_SHARED((NPAD,), jnp.float32),
        ],
    )
    def k(dst_hbm, out_hbm, dst_v, val_v, acc_sh):
        c = lax.axis_index("c")
        s = lax.axis_index("s")
        wid = c * ns + s
        zero16 = jnp.zeros((16,), jnp.float32)
        for j in range(128 // 16):
            val_v[pl.ds(j * 16, 16)] = zero16
        for j in range(rows_per_tile // 128):
            pltpu.sync_copy(val_v, acc_sh.at[pl.ds(s * rows_per_tile + j * 128, 128)])
        plsc.subcore_barrier()
        one16 = jnp.ones((16,), jnp.float32)
        for j in range(128 // 16):
            val_v[pl.ds(j * 16, 16)] = one16
        pltpu.sync_copy(dst_hbm.at[pl.ds(wid * rpw, rpw)], dst_v)
        for j in range(rpw):
            pltpu.sync_copy(val_v, acc_sh.at[dst_v.at[j]], add=True)
        plsc.subcore_barrier()
        pltpu.sync_copy(acc_sh.at[pl.ds(s * rows_per_tile, rows_per_tile)],
                        out_hbm.at[c, pl.ds(s * rows_per_tile, rows_per_tile)])

    return k(dstR)


def _sc_scatter(z0, z1, srcR, dstR, nc, ns, rpw):
    mesh = plsc.VectorSubcoreMesh(core_axis_name="c", subcore_axis_name="s")
    rows_per_tile = NPAD // ns

    nbuf = 4
    pipe = 3
    ich = 16
    rows_tot = rpw * nc * ns
    rows_pt = rows_tot // ns
    assert rows_pt % ich == 0

    @functools.partial(
        pl.kernel,
        out_type=[jax.ShapeDtypeStruct((NPAD, WA), jnp.float32),
                  jax.ShapeDtypeStruct((NPAD, WB), jnp.float32)],
        mesh=mesh,
        compiler_params=pltpu.CompilerParams(use_tc_tiling_on_sc=False),
        scratch_types=(
            [pltpu.VMEM((ich, 128), jnp.int32) for _ in range(2)]
            + [pltpu.VMEM((128, WA), jnp.float32) for _ in range(nbuf)]
            + [pltpu.VMEM_SHARED((NPAD, WA), jnp.float32)]
            + [pltpu.VMEM_SHARED((NPAD, WB), jnp.float32)]
            + [pltpu.SemaphoreType.DMA for _ in range(2 * nbuf)]
        ),
    )
    def k(z0_hbm, z1_hbm, src_hbm, dst_hbm, q0_hbm, q1_hbm, *rest):
        sidx, didx = rest[0:2]
        bufa = rest[2:2 + nbuf]
        bufb = bufa
        acc_a = rest[2 + nbuf]
        acc_b = rest[3 + nbuf]
        gsem = rest[4 + nbuf:4 + 2 * nbuf]
        ssem = rest[4 + 2 * nbuf:]
        c = lax.axis_index("c")
        s = lax.axis_index("s")
        zero16 = jnp.zeros((16,), jnp.float32)

        def zrow(i, carry):
            for j in range(WA // 16):
                bufa[0][i, pl.ds(j * 16, 16)] = zero16
            return carry

        lax.fori_loop(0, 128, zrow, 0)

        @pl.when(c == 0)
        def _():
            for j in range(rows_per_tile // 128):
                pltpu.sync_copy(
                    bufa[0], acc_a.at[pl.ds(s * rows_per_tile + j * 128, 128)])

        @pl.when(c == 1)
        def _():
            for j in range(rows_per_tile // 128):
                pltpu.sync_copy(
                    bufb[0], acc_b.at[pl.ds(s * rows_per_tile + j * 128, 128)])

        plsc.subcore_barrier()

        def chunk_body(row_base, z_hbm, bufs, acc_sh):
            r0 = pl.multiple_of(row_base, 8)
            pltpu.sync_copy(src_hbm.at[pl.ds(r0, ich)], sidx)
            pltpu.sync_copy(dst_hbm.at[pl.ds(r0, ich)], didx)
            gcp = [None] * nbuf
            scp = [None] * nbuf
            for r in range(ich + pipe):
                if r < ich:
                    b = r % nbuf
                    if r >= nbuf:
                        scp[b].wait()
                    gcp[b] = pltpu.async_copy(
                        z_hbm.at[sidx.at[r]], bufs[b], gsem[b])
                if r >= pipe:
                    i = r - pipe
                    bi = i % nbuf
                    gcp[bi].wait()
                    scp[bi] = pltpu.async_copy(
                        bufs[bi], acc_sh.at[didx.at[i]], ssem[bi], add=True)
            for i in range(ich - nbuf, ich):
                scp[i % nbuf].wait()

        def edge_loop(z_hbm, bufs, acc_sh):
            def body(ci, carry):
                chunk_body(s * rows_pt + ci * ich, z_hbm, bufs, acc_sh)
                return carry
            lax.fori_loop(0, rows_pt // ich, body, 0)

        @pl.when(c == 0)
        def _():
            edge_loop(z0_hbm, bufa, acc_a)

        @pl.when(c == 1)
        def _():
            edge_loop(z1_hbm, bufb, acc_b)

        plsc.subcore_barrier()

        @pl.when(c == 0)
        def _():
            for j in range(rows_per_tile // 128):
                r0 = s * rows_per_tile + j * 128
                pltpu.sync_copy(acc_a.at[pl.ds(r0, 128)],
                                q0_hbm.at[pl.ds(r0, 128)])

        @pl.when(c == 1)
        def _():
            for j in range(rows_per_tile // 128):
                r0 = s * rows_per_tile + j * 128
                pltpu.sync_copy(acc_b.at[pl.ds(r0, 128)],
                                q1_hbm.at[pl.ds(r0, 128)])

    return k(z0, z1, srcR, dstR)


def _split_specs():
    return [pl.BlockSpec((BLK, WA), lambda i: (i, 0)),
            pl.BlockSpec((BLK, WB), lambda i: (i, 0))]


def _split_shapes(n):
    return [jax.ShapeDtypeStruct((n, WA), jnp.float32),
            jax.ShapeDtypeStruct((n, WB), jnp.float32)]


def _dinv(cnt_ref):
    return lax.rsqrt(cnt_ref[0] + cnt_ref[1] + 1.0)


def _tc_first(x, W, cnt3):
    n, d = x.shape
    h = W.shape[1]

    def body(x_ref, w_ref, cnt_ref, z0_ref, z1_ref):
        xw = jnp.dot(x_ref[...], w_ref[...], preferred_element_type=jnp.float32)
        z = _dinv(cnt_ref) * xw
        z0_ref[...] = z[:, :WA]
        z1_ref[...] = z[:, WA:]

    return pl.pallas_call(
        body,
        grid=(n // BLK,),
        in_specs=[
            pl.BlockSpec((BLK, d), lambda i: (i, 0)),
            pl.BlockSpec((d, h), lambda i: (0, 0)),
            pl.BlockSpec((2, BLK, 1), lambda i: (0, i, 0)),
        ],
        out_specs=_split_specs(),
        out_shape=_split_shapes(n),
    )(x, W, cnt3)


def _combine(q0_ref, q1_ref, z0_ref, z1_ref, b_ref, dinv):
    h0 = dinv * (q0_ref[...] + z0_ref[...]) + b_ref[:, :WA]
    h1 = dinv * (q1_ref[...] + z1_ref[...]) + b_ref[:, WA:]
    return jnp.maximum(jnp.concatenate([h0, h1], axis=1), 0.0)


def _tc_mid(q, z, b, W, cnt3):
    q0, q1 = q
    z0, z1 = z
    n = q0.shape[0]
    h2 = W.shape[1]

    def body(q0_ref, q1_ref, z0_ref, z1_ref, b_ref, w_ref, cnt_ref,
             zo0_ref, zo1_ref):
        dinv = _dinv(cnt_ref)
        hh = _combine(q0_ref, q1_ref, z0_ref, z1_ref, b_ref, dinv)
        zn = dinv * jnp.dot(hh, w_ref[...],
                            preferred_element_type=jnp.float32)
        zo0_ref[...] = zn[:, :WA]
        zo1_ref[...] = zn[:, WA:]

    return pl.pallas_call(
        body,
        grid=(n // BLK,),
        in_specs=_split_specs() + _split_specs() + [
            pl.BlockSpec((1, 128), lambda i: (0, 0)),
            pl.BlockSpec((128, h2), lambda i: (0, 0)),
            pl.BlockSpec((2, BLK, 1), lambda i: (0, i, 0)),
        ],
        out_specs=_split_specs(),
        out_shape=_split_shapes(n),
    )(q0, q1, z0, z1, b, W, cnt3)


def _tc_final(q, z, b, cnt3, batch2, Wl_pad, bl_pad):
    q0, q1 = q
    z0, z1 = z
    n = q0.shape[0]
    nblk = n // BLK

    def body(q0_ref, q1_ref, z0_ref, z1_ref, b_ref, cnt_ref, bt_ref,
             wl_ref, bl_ref, out_ref, sums, counts):
        i = pl.program_id(0)

        @pl.when(i == 0)
        def _():
            sums[...] = jnp.zeros_like(sums)
            counts[...] = jnp.zeros_like(counts)

        hh = _combine(q0_ref, q1_ref, z0_ref, z1_ref, b_ref, _dinv(cnt_ref))
        onehot = (bt_ref[...] == lax.broadcasted_iota(
            jnp.int32, (BLK, GSEG), 1)).astype(jnp.float32)
        dn = (((0,), (0,)), ((), ()))
        sums[...] += lax.dot_general(onehot, hh, dn,
                                     preferred_element_type=jnp.float32)
        counts[...] += lax.dot_general(onehot, jnp.ones((BLK, 128), jnp.float32),
                                       dn, preferred_element_type=jnp.float32)

        @pl.when(i == nblk - 1)
        def _():
            pooled = sums[...] / jnp.maximum(counts[...], 1.0)
            out_ref[...] = jnp.dot(pooled, wl_ref[...],
                                   preferred_element_type=jnp.float32) + bl_ref[...]

    return pl.pallas_call(
        body,
        grid=(nblk,),
        in_specs=_split_specs() + _split_specs() + [
            pl.BlockSpec((1, 128), lambda i: (0, 0)),
            pl.BlockSpec((2, BLK, 1), lambda i: (0, i, 0)),
            pl.BlockSpec((BLK, 1), lambda i: (i, 0)),
            pl.BlockSpec((128, 128), lambda i: (0, 0)),
            pl.BlockSpec((1, 128), lambda i: (0, 0)),
        ],
        out_specs=pl.BlockSpec((GSEG, 128), lambda i: (0, 0)),
        out_shape=jax.ShapeDtypeStruct((GSEG, 128), jnp.float32),
        scratch_shapes=[
            pltpu.VMEM((GSEG, 128), jnp.float32),
            pltpu.VMEM((GSEG, 128), jnp.float32),
        ],
    )(q0, q1, z0, z1, b, cnt3, batch2, Wl_pad, bl_pad)


def kernel(x, edge_index, batch, W1, b1, W2, b2, W3, b3, Wl, bl):
    n, d = x.shape
    e = edge_index.shape[1]
    h = W1.shape[1]

    info = plsc.get_sparse_core_info()
    nc, ns = info.num_cores, info.num_subcores
    nw = nc * ns

    rows = -(-e // 128)
    rpw = -(-rows // nw)
    rpw = -(-rpw // 16) * 16
    rows_tot = rpw * nw
    pad_e = rows_tot * 128 - e

    x_pad = jnp.pad(x, ((0, NPAD - n), (0, 0)))
    fill = jnp.full((pad_e,), NPAD - 1, jnp.int32)
    srcR = jnp.concatenate([edge_index[0], fill]).reshape(rows_tot, 128)
    dstR = jnp.concatenate([edge_index[1], fill]).reshape(rows_tot, 128)
    batch2 = jnp.pad(batch, (0, NPAD - n), constant_values=GSEG).reshape(NPAD, 1)
    b1r = b1.reshape(1, h)
    b2r = b2.reshape(1, h)
    b3r = b3.reshape(1, h)
    Wl_pad = jnp.pad(Wl, ((0, 0), (0, 128 - Wl.shape[1])))
    bl_pad = jnp.pad(bl, (0, 128 - bl.shape[0])).reshape(1, 128)

    cnt3 = _sc_count(dstR, nc, ns, rpw).reshape(nc, NPAD, 1)

    z1 = _tc_first(x_pad, W1, cnt3)
    p1 = _sc_scatter(z1[0], z1[1], srcR, dstR, nc, ns, rpw)
    z2 = _tc_mid(p1, z1, b1r, W2, cnt3)
    p2 = _sc_scatter(z2[0], z2[1], srcR, dstR, nc, ns, rpw)
    z3 = _tc_mid(p2, z2, b2r, W3, cnt3)
    p3 = _sc_scatter(z3[0], z3[1], srcR, dstR, nc, ns, rpw)
    out128 = _tc_final(p3, z3, b3r, cnt3, batch2, Wl_pad, bl_pad)
    return out128[:, : Wl.shape[1]]


# speedup vs baseline: 1.1179x; 1.1179x over previous
import functools

import jax
import jax.numpy as jnp
from jax import lax
from jax.experimental import pallas as pl
from jax.experimental.pallas import tpu as pltpu
from jax.experimental.pallas import tpu_sc as plsc

NPAD = 10240
GSEG = 128
BLK = 512
WA = 64
WB = 128 - WA


def _sc_count(dstR, nc, ns, rpw):
    mesh = plsc.VectorSubcoreMesh(core_axis_name="c", subcore_axis_name="s")
    rows_per_tile = NPAD // ns

    @functools.partial(
        pl.kernel,
        out_type=jax.ShapeDtypeStruct((nc, NPAD), jnp.float32),
        mesh=mesh,
        scratch_types=[
            pltpu.VMEM((rpw, 128), jnp.int32),
            pltpu.VMEM((128,), jnp.float32),
            pltpu.VMEM_SHARED((NPAD,), jnp.float32),
        ],
    )
    def k(dst_hbm, out_hbm, dst_v, val_v, acc_sh):
        c = lax.axis_index("c")
        s = lax.axis_index("s")
        wid = c * ns + s
        zero16 = jnp.zeros((16,), jnp.float32)
        for j in range(128 // 16):
            val_v[pl.ds(j * 16, 16)] = zero16
        for j in range(rows_per_tile // 128):
            pltpu.sync_copy(val_v, acc_sh.at[pl.ds(s * rows_per_tile + j * 128, 128)])
        plsc.subcore_barrier()
        one16 = jnp.ones((16,), jnp.float32)
        for j in range(128 // 16):
            val_v[pl.ds(j * 16, 16)] = one16
        pltpu.sync_copy(dst_hbm.at[pl.ds(wid * rpw, rpw)], dst_v)
        for j in range(rpw):
            pltpu.sync_copy(val_v, acc_sh.at[dst_v.at[j]], add=True)
        plsc.subcore_barrier()
        pltpu.sync_copy(acc_sh.at[pl.ds(s * rows_per_tile, rows_per_tile)],
                        out_hbm.at[c, pl.ds(s * rows_per_tile, rows_per_tile)])

    return k(dstR)


def _sc_scatter(z0, z1, srcR, dstR, nc, ns, rpw):
    mesh = plsc.VectorSubcoreMesh(core_axis_name="c", subcore_axis_name="s")
    rows_per_tile = NPAD // ns

    nbuf = 4
    pipe = 3
    ich = 16
    rows_tot = rpw * nc * ns
    rows_pt = rows_tot // ns
    assert rows_pt % ich == 0

    @functools.partial(
        pl.kernel,
        out_type=[jax.ShapeDtypeStruct((NPAD, WA), jnp.float32),
                  jax.ShapeDtypeStruct((NPAD, WB), jnp.float32)],
        mesh=mesh,
        compiler_params=pltpu.CompilerParams(use_tc_tiling_on_sc=False),
        scratch_types=(
            [pltpu.VMEM((ich, 128), jnp.int32) for _ in range(2)]
            + [pltpu.VMEM((128, WA), jnp.float32) for _ in range(nbuf)]
            + [pltpu.VMEM_SHARED((NPAD, WA), jnp.float32)]
            + [pltpu.VMEM_SHARED((NPAD, WB), jnp.float32)]
            + [pltpu.SemaphoreType.DMA for _ in range(2 * nbuf)]
        ),
    )
    def k(z0_hbm, z1_hbm, src_hbm, dst_hbm, q0_hbm, q1_hbm, *rest):
        sidx, didx = rest[0:2]
        bufa = rest[2:2 + nbuf]
        bufb = bufa
        acc_a = rest[2 + nbuf]
        acc_b = rest[3 + nbuf]
        gsem = rest[4 + nbuf:4 + 2 * nbuf]
        ssem = rest[4 + 2 * nbuf:]
        c = lax.axis_index("c")
        s = lax.axis_index("s")
        zero16 = jnp.zeros((16,), jnp.float32)

        def zrow(i, carry):
            for j in range(WA // 16):
                bufa[0][i, pl.ds(j * 16, 16)] = zero16
            return carry

        lax.fori_loop(0, 128, zrow, 0)

        @pl.when(c == 0)
        def _():
            for j in range(rows_per_tile // 128):
                pltpu.sync_copy(
                    bufa[0], acc_a.at[pl.ds(s * rows_per_tile + j * 128, 128)])

        @pl.when(c == 1)
        def _():
            for j in range(rows_per_tile // 128):
                pltpu.sync_copy(
                    bufb[0], acc_b.at[pl.ds(s * rows_per_tile + j * 128, 128)])

        plsc.subcore_barrier()

        def chunk_body(row_base, z_hbm, bufs, acc_sh):
            r0 = pl.multiple_of(row_base, 8)
            pltpu.sync_copy(src_hbm.at[pl.ds(r0, ich)], sidx)
            pltpu.sync_copy(dst_hbm.at[pl.ds(r0, ich)], didx)
            gcp = [None] * nbuf
            scp = [None] * nbuf
            for r in range(ich + pipe):
                if r < ich:
                    b = r % nbuf
                    if r >= nbuf:
                        scp[b].wait()
                    gcp[b] = pltpu.async_copy(
                        z_hbm.at[sidx.at[r]], bufs[b], gsem[b])
                if r >= pipe:
                    i = r - pipe
                    bi = i % nbuf
                    gcp[bi].wait()
                    scp[bi] = pltpu.async_copy(
                        bufs[bi], acc_sh.at[didx.at[i]], ssem[bi], add=True)
            for i in range(ich - nbuf, ich):
                scp[i % nbuf].wait()

        def edge_loop(z_hbm, bufs, acc_sh):
            def body(ci, carry):
                chunk_body(s * rows_pt + ci * ich, z_hbm, bufs, acc_sh)
                return carry
            lax.fori_loop(0, rows_pt // ich, body, 0)

        @pl.when(c == 0)
        def _():
            edge_loop(z0_hbm, bufa, acc_a)

        @pl.when(c == 1)
        def _():
            edge_loop(z1_hbm, bufb, acc_b)

        plsc.subcore_barrier()

        @pl.when(c == 0)
        def _():
            for j in range(rows_per_tile // 128):
                r0 = s * rows_per_tile + j * 128
                pltpu.sync_copy(acc_a.at[pl.ds(r0, 128)],
                                q0_hbm.at[pl.ds(r0, 128)])

        @pl.when(c == 1)
        def _():
            for j in range(rows_per_tile // 128):
                r0 = s * rows_per_tile + j * 128
                pltpu.sync_copy(acc_b.at[pl.ds(r0, 128)],
                                q1_hbm.at[pl.ds(r0, 128)])

    return k(z0, z1, srcR, dstR)


def _split_specs():
    return [pl.BlockSpec((BLK, WA), lambda i: (i, 0)),
            pl.BlockSpec((BLK, WB), lambda i: (i, 0))]


def _split_shapes(n):
    return [jax.ShapeDtypeStruct((n, WA), jnp.float32),
            jax.ShapeDtypeStruct((n, WB), jnp.float32)]


def _tc_first(x, W, dinv):
    n, d = x.shape
    h = W.shape[1]

    def body(x_ref, w_ref, dinv_ref, z0_ref, z1_ref):
        xw = jnp.dot(x_ref[...], w_ref[...], preferred_element_type=jnp.float32)
        z = dinv_ref[...] * xw
        z0_ref[...] = z[:, :WA]
        z1_ref[...] = z[:, WA:]

    return pl.pallas_call(
        body,
        grid=(n // BLK,),
        in_specs=[
            pl.BlockSpec((BLK, d), lambda i: (i, 0)),
            pl.BlockSpec((d, h), lambda i: (0, 0)),
            pl.BlockSpec((BLK, 1), lambda i: (i, 0)),
        ],
        out_specs=_split_specs(),
        out_shape=_split_shapes(n),
    )(x, W, dinv)


def _combine(q0_ref, q1_ref, z0_ref, z1_ref, b_ref, dinv_ref):
    h0 = dinv_ref[...] * (q0_ref[...] + z0_ref[...]) + b_ref[:, :WA]
    h1 = dinv_ref[...] * (q1_ref[...] + z1_ref[...]) + b_ref[:, WA:]
    return jnp.maximum(jnp.concatenate([h0, h1], axis=1), 0.0)


def _tc_mid(q, z, b, W, dinv):
    q0, q1 = q
    z0, z1 = z
    n = q0.shape[0]
    h2 = W.shape[1]

    def body(q0_ref, q1_ref, z0_ref, z1_ref, b_ref, w_ref, dinv_ref,
             zo0_ref, zo1_ref):
        hh = _combine(q0_ref, q1_ref, z0_ref, z1_ref, b_ref, dinv_ref)
        zn = dinv_ref[...] * jnp.dot(hh, w_ref[...],
                                     preferred_element_type=jnp.float32)
        zo0_ref[...] = zn[:, :WA]
        zo1_ref[...] = zn[:, WA:]

    return pl.pallas_call(
        body,
        grid=(n // BLK,),
        in_specs=_split_specs() + _split_specs() + [
            pl.BlockSpec((1, 128), lambda i: (0, 0)),
            pl.BlockSpec((128, h2), lambda i: (0, 0)),
            pl.BlockSpec((BLK, 1), lambda i: (i, 0)),
        ],
        out_specs=_split_specs(),
        out_shape=_split_shapes(n),
    )(q0, q1, z0, z1, b, W, dinv)


def _tc_final(q, z, b, dinv, batch2, Wl_pad, bl_pad):
    q0, q1 = q
    z0, z1 = z
    n = q0.shape[0]
    nblk = n // BLK

    def body(q0_ref, q1_ref, z0_ref, z1_ref, b_ref, dinv_ref, bt_ref,
             wl_ref, bl_ref, out_ref, sums, counts):
        i = pl.program_id(0)

        @pl.when(i == 0)
        def _():
            sums[...] = jnp.zeros_like(sums)
            counts[...] = jnp.zeros_like(counts)

        hh = _combine(q0_ref, q1_ref, z0_ref, z1_ref, b_ref, dinv_ref)
        onehot = (bt_ref[...] == lax.broadcasted_iota(
            jnp.int32, (BLK, GSEG), 1)).astype(jnp.float32)
        dn = (((0,), (0,)), ((), ()))
        sums[...] += lax.dot_general(onehot, hh, dn,
                                     preferred_element_type=jnp.float32)
        counts[...] += lax.dot_general(onehot, jnp.ones((BLK, 128), jnp.float32),
                                       dn, preferred_element_type=jnp.float32)

        @pl.when(i == nblk - 1)
        def _():
            pooled = sums[...] / jnp.maximum(counts[...], 1.0)
            out_ref[...] = jnp.dot(pooled, wl_ref[...],
                                   preferred_element_type=jnp.float32) + bl_ref[...]

    return pl.pallas_call(
        body,
        grid=(nblk,),
        in_specs=_split_specs() + _split_specs() + [
            pl.BlockSpec((1, 128), lambda i: (0, 0)),
            pl.BlockSpec((BLK, 1), lambda i: (i, 0)),
            pl.BlockSpec((BLK, 1), lambda i: (i, 0)),
            pl.BlockSpec((128, 128), lambda i: (0, 0)),
            pl.BlockSpec((1, 128), lambda i: (0, 0)),
        ],
        out_specs=pl.BlockSpec((GSEG, 128), lambda i: (0, 0)),
        out_shape=jax.ShapeDtypeStruct((GSEG, 128), jnp.float32),
        scratch_shapes=[
            pltpu.VMEM((GSEG, 128), jnp.float32),
            pltpu.VMEM((GSEG, 128), jnp.float32),
        ],
    )(q0, q1, z0, z1, b, dinv, batch2, Wl_pad, bl_pad)


def kernel(x, edge_index, batch, W1, b1, W2, b2, W3, b3, Wl, bl):
    n, d = x.shape
    e = edge_index.shape[1]
    h = W1.shape[1]

    info = plsc.get_sparse_core_info()
    nc, ns = info.num_cores, info.num_subcores
    nw = nc * ns

    rows = -(-e // 128)
    rpw = -(-rows // nw)
    rpw = -(-rpw // 16) * 16
    rows_tot = rpw * nw
    pad_e = rows_tot * 128 - e

    x_pad = jnp.pad(x, ((0, NPAD - n), (0, 0)))
    fill = jnp.full((pad_e,), NPAD - 1, jnp.int32)
    srcR = jnp.concatenate([edge_index[0], fill]).reshape(rows_tot, 128)
    dstR = jnp.concatenate([edge_index[1], fill]).reshape(rows_tot, 128)
    batch2 = jnp.pad(batch, (0, NPAD - n), constant_values=GSEG).reshape(NPAD, 1)
    b1r = b1.reshape(1, h)
    b2r = b2.reshape(1, h)
    b3r = b3.reshape(1, h)
    Wl_pad = jnp.pad(Wl, ((0, 0), (0, 128 - Wl.shape[1])))
    bl_pad = jnp.pad(bl, (0, 128 - bl.shape[0])).reshape(1, 128)

    cnt = _sc_count(dstR, nc, ns, rpw)
    dinv = lax.rsqrt(cnt.sum(axis=0) + 1.0).reshape(NPAD, 1)

    z1 = _tc_first(x_pad, W1, dinv)
    p1 = _sc_scatter(z1[0], z1[1], srcR, dstR, nc, ns, rpw)
    z2 = _tc_mid(p1, z1, b1r, W2, dinv)
    p2 = _sc_scatter(z2[0], z2[1], srcR, dstR, nc, ns, rpw)
    z3 = _tc_mid(p2, z2, b2r, W3, dinv)
    p3 = _sc_scatter(z3[0], z3[1], srcR, dstR, nc, ns, rpw)
    out128 = _tc_final(p3, z3, b3r, dinv, batch2, Wl_pad, bl_pad)
    return out128[:, : Wl.shape[1]]


# speedup vs baseline: 1.1437x; 1.0231x over previous
import functools

import jax
import jax.numpy as jnp
from jax import lax
from jax.experimental import pallas as pl
from jax.experimental.pallas import tpu as pltpu
from jax.experimental.pallas import tpu_sc as plsc

NPAD = 10240
GSEG = 128
BLK = 1024
WA = 64
WB = 128 - WA


def _sc_count(dstR, nc, ns, rpw):
    mesh = plsc.VectorSubcoreMesh(core_axis_name="c", subcore_axis_name="s")
    rows_per_tile = NPAD // ns

    @functools.partial(
        pl.kernel,
        out_type=jax.ShapeDtypeStruct((nc, NPAD), jnp.float32),
        mesh=mesh,
        scratch_types=[
            pltpu.VMEM((rpw, 128), jnp.int32),
            pltpu.VMEM((128,), jnp.float32),
            pltpu.VMEM_SHARED((NPAD,), jnp.float32),
        ],
    )
    def k(dst_hbm, out_hbm, dst_v, val_v, acc_sh):
        c = lax.axis_index("c")
        s = lax.axis_index("s")
        wid = c * ns + s
        zero16 = jnp.zeros((16,), jnp.float32)
        for j in range(128 // 16):
            val_v[pl.ds(j * 16, 16)] = zero16
        for j in range(rows_per_tile // 128):
            pltpu.sync_copy(val_v, acc_sh.at[pl.ds(s * rows_per_tile + j * 128, 128)])
        plsc.subcore_barrier()
        one16 = jnp.ones((16,), jnp.float32)
        for j in range(128 // 16):
            val_v[pl.ds(j * 16, 16)] = one16
        pltpu.sync_copy(dst_hbm.at[pl.ds(wid * rpw, rpw)], dst_v)
        for j in range(rpw):
            pltpu.sync_copy(val_v, acc_sh.at[dst_v.at[j]], add=True)
        plsc.subcore_barrier()
        pltpu.sync_copy(acc_sh.at[pl.ds(s * rows_per_tile, rows_per_tile)],
                        out_hbm.at[c, pl.ds(s * rows_per_tile, rows_per_tile)])

    return k(dstR)


def _sc_scatter(z0, z1, srcR, dstR, nc, ns, rpw):
    mesh = plsc.VectorSubcoreMesh(core_axis_name="c", subcore_axis_name="s")
    rows_per_tile = NPAD // ns

    nbuf = 5
    pipe = 3
    ich = 16
    rows_tot = rpw * nc * ns
    rows_pt = rows_tot // ns
    assert rows_pt % ich == 0

    @functools.partial(
        pl.kernel,
        out_type=[jax.ShapeDtypeStruct((NPAD, WA), jnp.float32),
                  jax.ShapeDtypeStruct((NPAD, WB), jnp.float32)],
        mesh=mesh,
        compiler_params=pltpu.CompilerParams(use_tc_tiling_on_sc=False),
        scratch_types=(
            [pltpu.VMEM((ich, 128), jnp.int32) for _ in range(2)]
            + [pltpu.VMEM((128, WA), jnp.float32) for _ in range(nbuf)]
            + [pltpu.VMEM_SHARED((NPAD, WA), jnp.float32)]
            + [pltpu.VMEM_SHARED((NPAD, WB), jnp.float32)]
            + [pltpu.SemaphoreType.DMA for _ in range(2 * nbuf)]
        ),
    )
    def k(z0_hbm, z1_hbm, src_hbm, dst_hbm, q0_hbm, q1_hbm, *rest):
        sidx, didx = rest[0:2]
        bufa = rest[2:2 + nbuf]
        bufb = bufa
        acc_a = rest[2 + nbuf]
        acc_b = rest[3 + nbuf]
        gsem = rest[4 + nbuf:4 + 2 * nbuf]
        ssem = rest[4 + 2 * nbuf:]
        c = lax.axis_index("c")
        s = lax.axis_index("s")
        zero16 = jnp.zeros((16,), jnp.float32)

        def zrow(i, carry):
            for j in range(WA // 16):
                bufa[0][i, pl.ds(j * 16, 16)] = zero16
            return carry

        lax.fori_loop(0, 128, zrow, 0)

        @pl.when(c == 0)
        def _():
            for j in range(rows_per_tile // 128):
                pltpu.sync_copy(
                    bufa[0], acc_a.at[pl.ds(s * rows_per_tile + j * 128, 128)])

        @pl.when(c == 1)
        def _():
            for j in range(rows_per_tile // 128):
                pltpu.sync_copy(
                    bufb[0], acc_b.at[pl.ds(s * rows_per_tile + j * 128, 128)])

        plsc.subcore_barrier()

        def chunk_body(row_base, z_hbm, bufs, acc_sh):
            r0 = pl.multiple_of(row_base, 8)
            pltpu.sync_copy(src_hbm.at[pl.ds(r0, ich)], sidx)
            pltpu.sync_copy(dst_hbm.at[pl.ds(r0, ich)], didx)
            gcp = [None] * nbuf
            scp = [None] * nbuf
            for r in range(ich + pipe):
                if r < ich:
                    b = r % nbuf
                    if r >= nbuf:
                        scp[b].wait()
                    gcp[b] = pltpu.async_copy(
                        z_hbm.at[sidx.at[r]], bufs[b], gsem[b])
                if r >= pipe:
                    i = r - pipe
                    bi = i % nbuf
                    gcp[bi].wait()
                    scp[bi] = pltpu.async_copy(
                        bufs[bi], acc_sh.at[didx.at[i]], ssem[bi], add=True)
            for i in range(ich - nbuf, ich):
                scp[i % nbuf].wait()

        def edge_loop(z_hbm, bufs, acc_sh):
            def body(ci, carry):
                chunk_body(s * rows_pt + ci * ich, z_hbm, bufs, acc_sh)
                return carry
            lax.fori_loop(0, rows_pt // ich, body, 0)

        @pl.when(c == 0)
        def _():
            edge_loop(z0_hbm, bufa, acc_a)

        @pl.when(c == 1)
        def _():
            edge_loop(z1_hbm, bufb, acc_b)

        plsc.subcore_barrier()

        @pl.when(c == 0)
        def _():
            for j in range(rows_per_tile // 128):
                r0 = s * rows_per_tile + j * 128
                pltpu.sync_copy(acc_a.at[pl.ds(r0, 128)],
                                q0_hbm.at[pl.ds(r0, 128)])

        @pl.when(c == 1)
        def _():
            for j in range(rows_per_tile // 128):
                r0 = s * rows_per_tile + j * 128
                pltpu.sync_copy(acc_b.at[pl.ds(r0, 128)],
                                q1_hbm.at[pl.ds(r0, 128)])

    return k(z0, z1, srcR, dstR)


def _split_specs():
    return [pl.BlockSpec((BLK, WA), lambda i: (i, 0)),
            pl.BlockSpec((BLK, WB), lambda i: (i, 0))]


def _split_shapes(n):
    return [jax.ShapeDtypeStruct((n, WA), jnp.float32),
            jax.ShapeDtypeStruct((n, WB), jnp.float32)]


def _tc_first(x, W, dinv):
    n, d = x.shape
    h = W.shape[1]

    def body(x_ref, w_ref, dinv_ref, z0_ref, z1_ref):
        xw = jnp.dot(x_ref[...], w_ref[...], preferred_element_type=jnp.float32)
        z = dinv_ref[...] * xw
        z0_ref[...] = z[:, :WA]
        z1_ref[...] = z[:, WA:]

    return pl.pallas_call(
        body,
        grid=(n // BLK,),
        in_specs=[
            pl.BlockSpec((BLK, d), lambda i: (i, 0)),
            pl.BlockSpec((d, h), lambda i: (0, 0)),
            pl.BlockSpec((BLK, 1), lambda i: (i, 0)),
        ],
        out_specs=_split_specs(),
        out_shape=_split_shapes(n),
    )(x, W, dinv)


def _combine(q0_ref, q1_ref, z0_ref, z1_ref, b_ref, dinv_ref):
    h0 = dinv_ref[...] * (q0_ref[...] + z0_ref[...]) + b_ref[:, :WA]
    h1 = dinv_ref[...] * (q1_ref[...] + z1_ref[...]) + b_ref[:, WA:]
    return jnp.maximum(jnp.concatenate([h0, h1], axis=1), 0.0)


def _tc_mid(q, z, b, W, dinv):
    q0, q1 = q
    z0, z1 = z
    n = q0.shape[0]
    h2 = W.shape[1]

    def body(q0_ref, q1_ref, z0_ref, z1_ref, b_ref, w_ref, dinv_ref,
             zo0_ref, zo1_ref):
        hh = _combine(q0_ref, q1_ref, z0_ref, z1_ref, b_ref, dinv_ref)
        zn = dinv_ref[...] * jnp.dot(hh, w_ref[...],
                                     preferred_element_type=jnp.float32)
        zo0_ref[...] = zn[:, :WA]
        zo1_ref[...] = zn[:, WA:]

    return pl.pallas_call(
        body,
        grid=(n // BLK,),
        in_specs=_split_specs() + _split_specs() + [
            pl.BlockSpec((1, 128), lambda i: (0, 0)),
            pl.BlockSpec((128, h2), lambda i: (0, 0)),
            pl.BlockSpec((BLK, 1), lambda i: (i, 0)),
        ],
        out_specs=_split_specs(),
        out_shape=_split_shapes(n),
    )(q0, q1, z0, z1, b, W, dinv)


def _tc_final(q, z, b, dinv, batch2, Wl_pad, bl_pad):
    q0, q1 = q
    z0, z1 = z
    n = q0.shape[0]
    nblk = n // BLK

    def body(q0_ref, q1_ref, z0_ref, z1_ref, b_ref, dinv_ref, bt_ref,
             wl_ref, bl_ref, out_ref, sums, counts):
        i = pl.program_id(0)

        @pl.when(i == 0)
        def _():
            sums[...] = jnp.zeros_like(sums)
            counts[...] = jnp.zeros_like(counts)

        hh = _combine(q0_ref, q1_ref, z0_ref, z1_ref, b_ref, dinv_ref)
        onehot = (bt_ref[...] == lax.broadcasted_iota(
            jnp.int32, (BLK, GSEG), 1)).astype(jnp.float32)
        dn = (((0,), (0,)), ((), ()))
        sums[...] += lax.dot_general(onehot, hh, dn,
                                     preferred_element_type=jnp.float32)
        counts[...] += lax.dot_general(onehot, jnp.ones((BLK, 128), jnp.float32),
                                       dn, preferred_element_type=jnp.float32)

        @pl.when(i == nblk - 1)
        def _():
            pooled = sums[...] / jnp.maximum(counts[...], 1.0)
            out_ref[...] = jnp.dot(pooled, wl_ref[...],
                                   preferred_element_type=jnp.float32) + bl_ref[...]

    return pl.pallas_call(
        body,
        grid=(nblk,),
        in_specs=_split_specs() + _split_specs() + [
            pl.BlockSpec((1, 128), lambda i: (0, 0)),
            pl.BlockSpec((BLK, 1), lambda i: (i, 0)),
            pl.BlockSpec((BLK, 1), lambda i: (i, 0)),
            pl.BlockSpec((128, 128), lambda i: (0, 0)),
            pl.BlockSpec((1, 128), lambda i: (0, 0)),
        ],
        out_specs=pl.BlockSpec((GSEG, 128), lambda i: (0, 0)),
        out_shape=jax.ShapeDtypeStruct((GSEG, 128), jnp.float32),
        scratch_shapes=[
            pltpu.VMEM((GSEG, 128), jnp.float32),
            pltpu.VMEM((GSEG, 128), jnp.float32),
        ],
    )(q0, q1, z0, z1, b, dinv, batch2, Wl_pad, bl_pad)


def kernel(x, edge_index, batch, W1, b1, W2, b2, W3, b3, Wl, bl):
    n, d = x.shape
    e = edge_index.shape[1]
    h = W1.shape[1]

    info = plsc.get_sparse_core_info()
    nc, ns = info.num_cores, info.num_subcores
    nw = nc * ns

    rows = -(-e // 128)
    rpw = -(-rows // nw)
    rpw = -(-rpw // 16) * 16
    rows_tot = rpw * nw
    pad_e = rows_tot * 128 - e

    x_pad = jnp.pad(x, ((0, NPAD - n), (0, 0)))
    fill = jnp.full((pad_e,), NPAD - 1, jnp.int32)
    srcR = jnp.concatenate([edge_index[0], fill]).reshape(rows_tot, 128)
    dstR = jnp.concatenate([edge_index[1], fill]).reshape(rows_tot, 128)
    batch2 = jnp.pad(batch, (0, NPAD - n), constant_values=GSEG).reshape(NPAD, 1)
    b1r = b1.reshape(1, h)
    b2r = b2.reshape(1, h)
    b3r = b3.reshape(1, h)
    Wl_pad = jnp.pad(Wl, ((0, 0), (0, 128 - Wl.shape[1])))
    bl_pad = jnp.pad(bl, (0, 128 - bl.shape[0])).reshape(1, 128)

    cnt = _sc_count(dstR, nc, ns, rpw)
    dinv = lax.rsqrt(cnt.sum(axis=0) + 1.0).reshape(NPAD, 1)

    z1 = _tc_first(x_pad, W1, dinv)
    p1 = _sc_scatter(z1[0], z1[1], srcR, dstR, nc, ns, rpw)
    z2 = _tc_mid(p1, z1, b1r, W2, dinv)
    p2 = _sc_scatter(z2[0], z2[1], srcR, dstR, nc, ns, rpw)
    z3 = _tc_mid(p2, z2, b2r, W3, dinv)
    p3 = _sc_scatter(z3[0], z3[1], srcR, dstR, nc, ns, rpw)
    out128 = _tc_final(p3, z3, b3r, dinv, batch2, Wl_pad, bl_pad)
    return out128[:, : Wl.shape[1]]


# speedup vs baseline: 1.1474x; 1.0032x over previous
import functools

import jax
import jax.numpy as jnp
from jax import lax
from jax.experimental import pallas as pl
from jax.experimental.pallas import tpu as pltpu
from jax.experimental.pallas import tpu_sc as plsc

NPAD = 10240
GSEG = 128
BLK = 1024
WA = 64
WB = 128 - WA


def _sc_count(dstR, nc, ns, rpw):
    mesh = plsc.VectorSubcoreMesh(core_axis_name="c", subcore_axis_name="s")
    rows_per_tile = NPAD // ns

    @functools.partial(
        pl.kernel,
        out_type=jax.ShapeDtypeStruct((nc, NPAD), jnp.float32),
        mesh=mesh,
        scratch_types=[
            pltpu.VMEM((rpw, 128), jnp.int32),
            pltpu.VMEM((128,), jnp.float32),
            pltpu.VMEM_SHARED((NPAD,), jnp.float32),
        ],
    )
    def k(dst_hbm, out_hbm, dst_v, val_v, acc_sh):
        c = lax.axis_index("c")
        s = lax.axis_index("s")
        wid = c * ns + s
        zero16 = jnp.zeros((16,), jnp.float32)
        for j in range(128 // 16):
            val_v[pl.ds(j * 16, 16)] = zero16
        for j in range(rows_per_tile // 128):
            pltpu.sync_copy(val_v, acc_sh.at[pl.ds(s * rows_per_tile + j * 128, 128)])
        plsc.subcore_barrier()
        one16 = jnp.ones((16,), jnp.float32)
        for j in range(128 // 16):
            val_v[pl.ds(j * 16, 16)] = one16
        pltpu.sync_copy(dst_hbm.at[pl.ds(wid * rpw, rpw)], dst_v)
        for j in range(rpw):
            pltpu.sync_copy(val_v, acc_sh.at[dst_v.at[j]], add=True)
        plsc.subcore_barrier()
        pltpu.sync_copy(acc_sh.at[pl.ds(s * rows_per_tile, rows_per_tile)],
                        out_hbm.at[c, pl.ds(s * rows_per_tile, rows_per_tile)])

    return k(dstR)


def _sc_scatter(z0, z1, srcR, dstR, nc, ns, rpw):
    mesh = plsc.VectorSubcoreMesh(core_axis_name="c", subcore_axis_name="s")
    rows_per_tile = NPAD // ns

    nbuf = 5
    pipe = 4
    ich = 16
    rows_tot = rpw * nc * ns
    rows_pt = rows_tot // ns
    assert rows_pt % ich == 0

    @functools.partial(
        pl.kernel,
        out_type=[jax.ShapeDtypeStruct((NPAD, WA), jnp.float32),
                  jax.ShapeDtypeStruct((NPAD, WB), jnp.float32)],
        mesh=mesh,
        compiler_params=pltpu.CompilerParams(use_tc_tiling_on_sc=False),
        scratch_types=(
            [pltpu.VMEM((ich, 128), jnp.int32) for _ in range(2)]
            + [pltpu.VMEM((128, WA), jnp.float32) for _ in range(nbuf)]
            + [pltpu.VMEM_SHARED((NPAD, WA), jnp.float32)]
            + [pltpu.VMEM_SHARED((NPAD, WB), jnp.float32)]
            + [pltpu.SemaphoreType.DMA for _ in range(2 * nbuf)]
        ),
    )
    def k(z0_hbm, z1_hbm, src_hbm, dst_hbm, q0_hbm, q1_hbm, *rest):
        sidx, didx = rest[0:2]
        bufa = rest[2:2 + nbuf]
        bufb = bufa
        acc_a = rest[2 + nbuf]
        acc_b = rest[3 + nbuf]
        gsem = rest[4 + nbuf:4 + 2 * nbuf]
        ssem = rest[4 + 2 * nbuf:]
        c = lax.axis_index("c")
        s = lax.axis_index("s")
        zero16 = jnp.zeros((16,), jnp.float32)

        def zrow(i, carry):
            for j in range(WA // 16):
                bufa[0][i, pl.ds(j * 16, 16)] = zero16
            return carry

        lax.fori_loop(0, 128, zrow, 0)

        @pl.when(c == 0)
        def _():
            for j in range(rows_per_tile // 128):
                pltpu.sync_copy(
                    bufa[0], acc_a.at[pl.ds(s * rows_per_tile + j * 128, 128)])

        @pl.when(c == 1)
        def _():
            for j in range(rows_per_tile // 128):
                pltpu.sync_copy(
                    bufb[0], acc_b.at[pl.ds(s * rows_per_tile + j * 128, 128)])

        plsc.subcore_barrier()

        def chunk_body(row_base, z_hbm, bufs, acc_sh):
            r0 = pl.multiple_of(row_base, 8)
            pltpu.sync_copy(src_hbm.at[pl.ds(r0, ich)], sidx)
            pltpu.sync_copy(dst_hbm.at[pl.ds(r0, ich)], didx)
            gcp = [None] * nbuf
            scp = [None] * nbuf
            for r in range(ich + pipe):
                if r < ich:
                    b = r % nbuf
                    if r >= nbuf:
                        scp[b].wait()
                    gcp[b] = pltpu.async_copy(
                        z_hbm.at[sidx.at[r]], bufs[b], gsem[b])
                if r >= pipe:
                    i = r - pipe
                    bi = i % nbuf
                    gcp[bi].wait()
                    scp[bi] = pltpu.async_copy(
                        bufs[bi], acc_sh.at[didx.at[i]], ssem[bi], add=True)
            for i in range(ich - nbuf, ich):
                scp[i % nbuf].wait()

        def edge_loop(z_hbm, bufs, acc_sh):
            def body(ci, carry):
                chunk_body(s * rows_pt + ci * ich, z_hbm, bufs, acc_sh)
                return carry
            lax.fori_loop(0, rows_pt // ich, body, 0)

        @pl.when(c == 0)
        def _():
            edge_loop(z0_hbm, bufa, acc_a)

        @pl.when(c == 1)
        def _():
            edge_loop(z1_hbm, bufb, acc_b)

        plsc.subcore_barrier()

        @pl.when(c == 0)
        def _():
            for j in range(rows_per_tile // 128):
                r0 = s * rows_per_tile + j * 128
                pltpu.sync_copy(acc_a.at[pl.ds(r0, 128)],
                                q0_hbm.at[pl.ds(r0, 128)])

        @pl.when(c == 1)
        def _():
            for j in range(rows_per_tile // 128):
                r0 = s * rows_per_tile + j * 128
                pltpu.sync_copy(acc_b.at[pl.ds(r0, 128)],
                                q1_hbm.at[pl.ds(r0, 128)])

    return k(z0, z1, srcR, dstR)


def _split_specs():
    return [pl.BlockSpec((BLK, WA), lambda i: (i, 0)),
            pl.BlockSpec((BLK, WB), lambda i: (i, 0))]


def _split_shapes(n):
    return [jax.ShapeDtypeStruct((n, WA), jnp.float32),
            jax.ShapeDtypeStruct((n, WB), jnp.float32)]


def _tc_first(x, W, dinv):
    n, d = x.shape
    h = W.shape[1]

    def body(x_ref, w_ref, dinv_ref, z0_ref, z1_ref):
        xw = jnp.dot(x_ref[...], w_ref[...], preferred_element_type=jnp.float32)
        z = dinv_ref[...] * xw
        z0_ref[...] = z[:, :WA]
        z1_ref[...] = z[:, WA:]

    return pl.pallas_call(
        body,
        grid=(n // BLK,),
        in_specs=[
            pl.BlockSpec((BLK, d), lambda i: (i, 0)),
            pl.BlockSpec((d, h), lambda i: (0, 0)),
            pl.BlockSpec((BLK, 1), lambda i: (i, 0)),
        ],
        out_specs=_split_specs(),
        out_shape=_split_shapes(n),
    )(x, W, dinv)


def _combine(q0_ref, q1_ref, z0_ref, z1_ref, b_ref, dinv_ref):
    h0 = dinv_ref[...] * (q0_ref[...] + z0_ref[...]) + b_ref[:, :WA]
    h1 = dinv_ref[...] * (q1_ref[...] + z1_ref[...]) + b_ref[:, WA:]
    return jnp.maximum(jnp.concatenate([h0, h1], axis=1), 0.0)


def _tc_mid(q, z, b, W, dinv):
    q0, q1 = q
    z0, z1 = z
    n = q0.shape[0]
    h2 = W.shape[1]

    def body(q0_ref, q1_ref, z0_ref, z1_ref, b_ref, w_ref, dinv_ref,
             zo0_ref, zo1_ref):
        hh = _combine(q0_ref, q1_ref, z0_ref, z1_ref, b_ref, dinv_ref)
        zn = dinv_ref[...] * jnp.dot(hh, w_ref[...],
                                     preferred_element_type=jnp.float32)
        zo0_ref[...] = zn[:, :WA]
        zo1_ref[...] = zn[:, WA:]

    return pl.pallas_call(
        body,
        grid=(n // BLK,),
        in_specs=_split_specs() + _split_specs() + [
            pl.BlockSpec((1, 128), lambda i: (0, 0)),
            pl.BlockSpec((128, h2), lambda i: (0, 0)),
            pl.BlockSpec((BLK, 1), lambda i: (i, 0)),
        ],
        out_specs=_split_specs(),
        out_shape=_split_shapes(n),
    )(q0, q1, z0, z1, b, W, dinv)


def _tc_final(q, z, b, dinv, batch2, Wl_pad, bl_pad):
    q0, q1 = q
    z0, z1 = z
    n = q0.shape[0]
    nblk = n // BLK

    def body(q0_ref, q1_ref, z0_ref, z1_ref, b_ref, dinv_ref, bt_ref,
             wl_ref, bl_ref, out_ref, sums, counts):
        i = pl.program_id(0)

        @pl.when(i == 0)
        def _():
            sums[...] = jnp.zeros_like(sums)
            counts[...] = jnp.zeros_like(counts)

        hh = _combine(q0_ref, q1_ref, z0_ref, z1_ref, b_ref, dinv_ref)
        onehot = (bt_ref[...] == lax.broadcasted_iota(
            jnp.int32, (BLK, GSEG), 1)).astype(jnp.float32)
        dn = (((0,), (0,)), ((), ()))
        sums[...] += lax.dot_general(onehot, hh, dn,
                                     preferred_element_type=jnp.float32)
        counts[...] += lax.dot_general(onehot, jnp.ones((BLK, 128), jnp.float32),
                                       dn, preferred_element_type=jnp.float32)

        @pl.when(i == nblk - 1)
        def _():
            pooled = sums[...] / jnp.maximum(counts[...], 1.0)
            out_ref[...] = jnp.dot(pooled, wl_ref[...],
                                   preferred_element_type=jnp.float32) + bl_ref[...]

    return pl.pallas_call(
        body,
        grid=(nblk,),
        in_specs=_split_specs() + _split_specs() + [
            pl.BlockSpec((1, 128), lambda i: (0, 0)),
            pl.BlockSpec((BLK, 1), lambda i: (i, 0)),
            pl.BlockSpec((BLK, 1), lambda i: (i, 0)),
            pl.BlockSpec((128, 128), lambda i: (0, 0)),
            pl.BlockSpec((1, 128), lambda i: (0, 0)),
        ],
        out_specs=pl.BlockSpec((GSEG, 128), lambda i: (0, 0)),
        out_shape=jax.ShapeDtypeStruct((GSEG, 128), jnp.float32),
        scratch_shapes=[
            pltpu.VMEM((GSEG, 128), jnp.float32),
            pltpu.VMEM((GSEG, 128), jnp.float32),
        ],
    )(q0, q1, z0, z1, b, dinv, batch2, Wl_pad, bl_pad)


def kernel(x, edge_index, batch, W1, b1, W2, b2, W3, b3, Wl, bl):
    n, d = x.shape
    e = edge_index.shape[1]
    h = W1.shape[1]

    info = plsc.get_sparse_core_info()
    nc, ns = info.num_cores, info.num_subcores
    nw = nc * ns

    rows = -(-e // 128)
    rpw = -(-rows // nw)
    rpw = -(-rpw // 16) * 16
    rows_tot = rpw * nw
    pad_e = rows_tot * 128 - e

    x_pad = jnp.pad(x, ((0, NPAD - n), (0, 0)))
    fill = jnp.full((pad_e,), NPAD - 1, jnp.int32)
    srcR = jnp.concatenate([edge_index[0], fill]).reshape(rows_tot, 128)
    dstR = jnp.concatenate([edge_index[1], fill]).reshape(rows_tot, 128)
    batch2 = jnp.pad(batch, (0, NPAD - n), constant_values=GSEG).reshape(NPAD, 1)
    b1r = b1.reshape(1, h)
    b2r = b2.reshape(1, h)
    b3r = b3.reshape(1, h)
    Wl_pad = jnp.pad(Wl, ((0, 0), (0, 128 - Wl.shape[1])))
    bl_pad = jnp.pad(bl, (0, 128 - bl.shape[0])).reshape(1, 128)

    cnt = _sc_count(dstR, nc, ns, rpw)
    dinv = lax.rsqrt(cnt.sum(axis=0) + 1.0).reshape(NPAD, 1)

    z1 = _tc_first(x_pad, W1, dinv)
    p1 = _sc_scatter(z1[0], z1[1], srcR, dstR, nc, ns, rpw)
    z2 = _tc_mid(p1, z1, b1r, W2, dinv)
    p2 = _sc_scatter(z2[0], z2[1], srcR, dstR, nc, ns, rpw)
    z3 = _tc_mid(p2, z2, b2r, W3, dinv)
    p3 = _sc_scatter(z3[0], z3[1], srcR, dstR, nc, ns, rpw)
    out128 = _tc_final(p3, z3, b3r, dinv, batch2, Wl_pad, bl_pad)
    return out128[:, : Wl.shape[1]]
